# baseline (speedup 1.0000x reference)
import functools

import jax
import jax.numpy as jnp
from jax import lax
from jax.experimental import pallas as pl
from jax.experimental.pallas import tpu as pltpu
from jax.experimental.pallas import tpu_sc as plsc

B = 16384
DIM = 64
TEXT_DIM = 768
NTAB = 1000

NC = 2
NS = 16
NW = NC * NS

NSPLIT = 4
BSUB = B // NSPLIT
ROWS_W = BSUB // NW

QCH = 64
NQC = ROWS_W // QCH

BLK = 1024


def _sc_gather(prompt, q_tab):
  mesh = plsc.VectorSubcoreMesh(core_axis_name="c", subcore_axis_name="s")

  @functools.partial(
      pl.kernel,
      mesh=mesh,
      out_type=jax.ShapeDtypeStruct((BSUB, TEXT_DIM), jnp.float32),
      scratch_types=[
          pltpu.VMEM((ROWS_W,), jnp.int32),
          pltpu.VMEM((QCH, TEXT_DIM), jnp.float32),
          pltpu.VMEM((QCH, TEXT_DIM), jnp.float32),
          pltpu.SemaphoreType.DMA,
          pltpu.SemaphoreType.DMA,
          pltpu.SemaphoreType.DMA,
          pltpu.SemaphoreType.DMA,
      ],
  )
  def k(prompt_hbm, qt_hbm, qraw_out, qidx, qbuf0, qbuf1,
        gsem0, gsem1, wsem0, wsem1):
    cid = lax.axis_index("c")
    sid = lax.axis_index("s")
    wid = sid * NC + cid
    base = wid * ROWS_W
    pltpu.sync_copy(prompt_hbm.at[pl.ds(base, ROWS_W)], qidx)

    qbufs = (qbuf0, qbuf1)
    gsems = (gsem0, gsem1)
    wsems = (wsem0, wsem1)

    gcp = {}
    wcp = {}
    gcp[0] = pltpu.async_copy(
        qt_hbm.at[qidx.at[pl.ds(0, QCH)]], qbufs[0], gsems[0])
    for c in range(NQC):
      b = c % 2
      gcp[c].wait()
      wcp[c] = pltpu.async_copy(
          qbufs[b], qraw_out.at[pl.ds(base + c * QCH, QCH)], wsems[b])
      if c + 1 < NQC:
        if c >= 1:
          wcp[c - 1].wait()
        gcp[c + 1] = pltpu.async_copy(
            qt_hbm.at[qidx.at[pl.ds((c + 1) * QCH, QCH)]],
            qbufs[1 - b], gsems[1 - b])
    wcp[NQC - 2].wait()
    wcp[NQC - 1].wait()

  return k(prompt, q_tab)


def _tc_dense(qraw, mid, cid, p_tab, c_tab, w_text, b_text, w_cls, b_cls):
  nblk = BSUB // BLK

  def body(q_ref, m_ref, c_ref, pt_ref, ct_ref, wt_ref, bt_ref, wc_ref,
           bc_ref, out_ref):
    q = jnp.dot(q_ref[...], wt_ref[...],
                preferred_element_type=jnp.float32) + bt_ref[...]
    iota = lax.broadcasted_iota(jnp.int32, (BLK, NTAB), 1)
    oh_m = (m_ref[...] == iota).astype(jnp.float32)
    oh_c = (c_ref[...] == iota).astype(jnp.float32)
    p = jnp.dot(oh_m, pt_ref[...], preferred_element_type=jnp.float32)
    v = jnp.dot(oh_c, ct_ref[...], preferred_element_type=jnp.float32)
    h = q * (p + v) + p * v
    out_ref[...] = jnp.dot(h, wc_ref[...],
                           preferred_element_type=jnp.float32) + bc_ref[...]

  return pl.pallas_call(
      body,
      grid=(nblk,),
      in_specs=[
          pl.BlockSpec((BLK, TEXT_DIM), lambda i: (i, 0)),
          pl.BlockSpec((BLK, 1), lambda i: (i, 0)),
          pl.BlockSpec((BLK, 1), lambda i: (i, 0)),
          pl.BlockSpec((NTAB, DIM), lambda i: (0, 0)),
          pl.BlockSpec((NTAB, DIM), lambda i: (0, 0)),
          pl.BlockSpec((TEXT_DIM, DIM), lambda i: (0, 0)),
          pl.BlockSpec((1, DIM), lambda i: (0, 0)),
          pl.BlockSpec((DIM, 2), lambda i: (0, 0)),
          pl.BlockSpec((1, 2), lambda i: (0, 0)),
      ],
      out_specs=pl.BlockSpec((BLK, 2), lambda i: (i, 0)),
      out_shape=jax.ShapeDtypeStruct((BSUB, 2), jnp.float32),
  )(qraw, mid, cid, p_tab, c_tab, w_text, b_text, w_cls, b_cls)


def kernel(model, prompt, category, P, Q, W_text, b_text, cat_emb, W_cls,
           b_cls):
  mid = model.reshape(B, 1)
  cid = category.reshape(B, 1)
  bt = b_text.reshape(1, DIM)
  bc = b_cls.reshape(1, 2)
  qraws = [
      _sc_gather(prompt[i * BSUB:(i + 1) * BSUB], Q) for i in range(NSPLIT)
  ]
  outs = [
      _tc_dense(qraws[i], mid[i * BSUB:(i + 1) * BSUB],
                cid[i * BSUB:(i + 1) * BSUB], P, cat_emb, W_text, bt,
                W_cls, bc) for i in range(NSPLIT)
  ]
  return jnp.concatenate(outs, axis=0)


# speedup vs baseline: 1.0297x; 1.0297x over previous
import functools

import jax
import jax.numpy as jnp
from jax import lax
from jax.experimental import pallas as pl
from jax.experimental.pallas import tpu as pltpu
from jax.experimental.pallas import tpu_sc as plsc

B = 16384
DIM = 64
TEXT_DIM = 768
NTAB = 1000

NC = 2
NS = 16
NW = NC * NS

NSPLIT = 2
BSUB = B // NSPLIT
ROWS_W = BSUB // NW

QCH = 64
NQC = ROWS_W // QCH

BLK = 1024


def _sc_gather(prompt, q_tab):
  mesh = plsc.VectorSubcoreMesh(core_axis_name="c", subcore_axis_name="s")

  @functools.partial(
      pl.kernel,
      mesh=mesh,
      out_type=jax.ShapeDtypeStruct((BSUB, TEXT_DIM), jnp.float32),
      scratch_types=[
          pltpu.VMEM((ROWS_W,), jnp.int32),
          pltpu.VMEM((QCH, TEXT_DIM), jnp.float32),
          pltpu.VMEM((QCH, TEXT_DIM), jnp.float32),
          pltpu.SemaphoreType.DMA,
          pltpu.SemaphoreType.DMA,
          pltpu.SemaphoreType.DMA,
          pltpu.SemaphoreType.DMA,
      ],
  )
  def k(prompt_hbm, qt_hbm, qraw_out, qidx, qbuf0, qbuf1,
        gsem0, gsem1, wsem0, wsem1):
    cid = lax.axis_index("c")
    sid = lax.axis_index("s")
    wid = sid * NC + cid
    base = wid * ROWS_W
    pltpu.sync_copy(prompt_hbm.at[pl.ds(base, ROWS_W)], qidx)

    qbufs = (qbuf0, qbuf1)
    gsems = (gsem0, gsem1)
    wsems = (wsem0, wsem1)

    gcp = {}
    wcp = {}
    gcp[0] = pltpu.async_copy(
        qt_hbm.at[qidx.at[pl.ds(0, QCH)]], qbufs[0], gsems[0])
    for c in range(NQC):
      b = c % 2
      gcp[c].wait()
      wcp[c] = pltpu.async_copy(
          qbufs[b], qraw_out.at[pl.ds(base + c * QCH, QCH)], wsems[b])
      if c + 1 < NQC:
        if c >= 1:
          wcp[c - 1].wait()
        gcp[c + 1] = pltpu.async_copy(
            qt_hbm.at[qidx.at[pl.ds((c + 1) * QCH, QCH)]],
            qbufs[1 - b], gsems[1 - b])
    wcp[NQC - 2].wait()
    wcp[NQC - 1].wait()

  return k(prompt, q_tab)


def _tc_dense(qraw, mid, cid, p_tab, c_tab, w_text, b_text, w_cls, b_cls):
  nblk = BSUB // BLK

  def body(q_ref, m_ref, c_ref, pt_ref, ct_ref, wt_ref, bt_ref, wc_ref,
           bc_ref, out_ref):
    q = jnp.dot(q_ref[...], wt_ref[...],
                preferred_element_type=jnp.float32) + bt_ref[...]
    iota = lax.broadcasted_iota(jnp.int16, (BLK, NTAB), 1)
    oh_m = (m_ref[...] == iota).astype(jnp.float32)
    oh_c = (c_ref[...] == iota).astype(jnp.float32)
    p = jnp.dot(oh_m, pt_ref[...], preferred_element_type=jnp.float32)
    v = jnp.dot(oh_c, ct_ref[...], preferred_element_type=jnp.float32)
    h = q * (p + v) + p * v
    out_ref[...] = jnp.dot(h, wc_ref[...],
                           preferred_element_type=jnp.float32) + bc_ref[...]

  return pl.pallas_call(
      body,
      grid=(nblk,),
      in_specs=[
          pl.BlockSpec((BLK, TEXT_DIM), lambda i: (i, 0)),
          pl.BlockSpec((BLK, 1), lambda i: (i, 0)),
          pl.BlockSpec((BLK, 1), lambda i: (i, 0)),
          pl.BlockSpec((NTAB, DIM), lambda i: (0, 0)),
          pl.BlockSpec((NTAB, DIM), lambda i: (0, 0)),
          pl.BlockSpec((TEXT_DIM, DIM), lambda i: (0, 0)),
          pl.BlockSpec((1, DIM), lambda i: (0, 0)),
          pl.BlockSpec((DIM, 2), lambda i: (0, 0)),
          pl.BlockSpec((1, 2), lambda i: (0, 0)),
      ],
      out_specs=pl.BlockSpec((BLK, 2), lambda i: (i, 0)),
      out_shape=jax.ShapeDtypeStruct((BSUB, 2), jnp.float32),
  )(qraw, mid, cid, p_tab, c_tab, w_text, b_text, w_cls, b_cls)


def kernel(model, prompt, category, P, Q, W_text, b_text, cat_emb, W_cls,
           b_cls):
  mid = model.astype(jnp.int16).reshape(B, 1)
  cid = category.astype(jnp.int16).reshape(B, 1)
  bt = b_text.reshape(1, DIM)
  bc = b_cls.reshape(1, 2)
  qraws = [
      _sc_gather(prompt[i * BSUB:(i + 1) * BSUB], Q) for i in range(NSPLIT)
  ]
  outs = [
      _tc_dense(qraws[i], mid[i * BSUB:(i + 1) * BSUB],
                cid[i * BSUB:(i + 1) * BSUB], P, cat_emb, W_text, bt,
                W_cls, bc) for i in range(NSPLIT)
  ]
  return jnp.concatenate(outs, axis=0)


# speedup vs baseline: 1.1021x; 1.0703x over previous
import functools

import jax
import jax.numpy as jnp
from jax import lax
from jax.experimental import pallas as pl
from jax.experimental.pallas import tpu as pltpu
from jax.experimental.pallas import tpu_sc as plsc

B = 16384
DIM = 64
TEXT_DIM = 768
NTAB = 1000

NC = 2
NS = 16
NW = NC * NS

NSPLIT = 2
BSUB = B // NSPLIT
ROWS_W = BSUB // NW

QCH = 64
NQC = ROWS_W // QCH

BLK = 1024


def _sc_gather(prompt, q_tab):
  mesh = plsc.VectorSubcoreMesh(core_axis_name="c", subcore_axis_name="s")

  @functools.partial(
      pl.kernel,
      mesh=mesh,
      out_type=jax.ShapeDtypeStruct((BSUB, TEXT_DIM), jnp.float32),
      scratch_types=[
          pltpu.VMEM((ROWS_W,), jnp.int32),
          pltpu.VMEM((QCH, TEXT_DIM), jnp.float32),
          pltpu.VMEM((QCH, TEXT_DIM), jnp.float32),
          pltpu.SemaphoreType.DMA,
          pltpu.SemaphoreType.DMA,
          pltpu.SemaphoreType.DMA,
          pltpu.SemaphoreType.DMA,
      ],
  )
  def k(prompt_hbm, qt_hbm, qraw_out, qidx, qbuf0, qbuf1,
        gsem0, gsem1, wsem0, wsem1):
    cid = lax.axis_index("c")
    sid = lax.axis_index("s")
    wid = sid * NC + cid
    base = wid * ROWS_W
    pltpu.sync_copy(prompt_hbm.at[pl.ds(base, ROWS_W)], qidx)

    qbufs = (qbuf0, qbuf1)
    gsems = (gsem0, gsem1)
    wsems = (wsem0, wsem1)

    gcp = {}
    wcp = {}
    gcp[0] = pltpu.async_copy(
        qt_hbm.at[qidx.at[pl.ds(0, QCH)]], qbufs[0], gsems[0])
    for c in range(NQC):
      b = c % 2
      gcp[c].wait()
      wcp[c] = pltpu.async_copy(
          qbufs[b], qraw_out.at[pl.ds(base + c * QCH, QCH)], wsems[b])
      if c + 1 < NQC:
        if c >= 1:
          wcp[c - 1].wait()
        gcp[c + 1] = pltpu.async_copy(
            qt_hbm.at[qidx.at[pl.ds((c + 1) * QCH, QCH)]],
            qbufs[1 - b], gsems[1 - b])
    wcp[NQC - 2].wait()
    wcp[NQC - 1].wait()

  return k(prompt, q_tab)


def _tc_dense(qraw, mid, cid, p_tab, c_tab, w_text, b_text, w_cls, b_cls):
  nblk = BSUB // BLK

  def body(q_ref, m_ref, c_ref, pt_ref, ct_ref, wt_ref, bt_ref, wc_ref,
           bc_ref, out_ref):
    q = jnp.dot(q_ref[...], wt_ref[...],
                preferred_element_type=jnp.float32) + bt_ref[...]
    iota = lax.broadcasted_iota(jnp.int32, (BLK, NTAB), 1)
    oh_m = (m_ref[...] == iota).astype(jnp.float32)
    oh_c = (c_ref[...] == iota).astype(jnp.float32)
    p = jnp.dot(oh_m, pt_ref[...], preferred_element_type=jnp.float32)
    v = jnp.dot(oh_c, ct_ref[...], preferred_element_type=jnp.float32)
    h = q * (p + v) + p * v
    out_ref[...] = jnp.dot(h, wc_ref[...],
                           preferred_element_type=jnp.float32) + bc_ref[...]

  return pl.pallas_call(
      body,
      grid=(nblk,),
      in_specs=[
          pl.BlockSpec((BLK, TEXT_DIM), lambda i: (i, 0)),
          pl.BlockSpec((BLK, 1), lambda i: (i, 0)),
          pl.BlockSpec((BLK, 1), lambda i: (i, 0)),
          pl.BlockSpec((NTAB, DIM), lambda i: (0, 0)),
          pl.BlockSpec((NTAB, DIM), lambda i: (0, 0)),
          pl.BlockSpec((TEXT_DIM, DIM), lambda i: (0, 0)),
          pl.BlockSpec((1, DIM), lambda i: (0, 0)),
          pl.BlockSpec((DIM, 2), lambda i: (0, 0)),
          pl.BlockSpec((1, 2), lambda i: (0, 0)),
      ],
      out_specs=pl.BlockSpec((BLK, 2), lambda i: (i, 0)),
      out_shape=jax.ShapeDtypeStruct((BSUB, 2), jnp.float32),
  )(qraw, mid, cid, p_tab, c_tab, w_text, b_text, w_cls, b_cls)


def kernel(model, prompt, category, P, Q, W_text, b_text, cat_emb, W_cls,
           b_cls):
  mid = model.reshape(B, 1)
  cid = category.reshape(B, 1)
  bt = b_text.reshape(1, DIM)
  bc = b_cls.reshape(1, 2)
  qraws = [
      _sc_gather(prompt[i * BSUB:(i + 1) * BSUB], Q) for i in range(NSPLIT)
  ]
  outs = [
      _tc_dense(qraws[i], mid[i * BSUB:(i + 1) * BSUB],
                cid[i * BSUB:(i + 1) * BSUB], P, cat_emb, W_text, bt,
                W_cls, bc) for i in range(NSPLIT)
  ]
  return jnp.concatenate(outs, axis=0)
